# baseline (speedup 1.0000x reference)
import functools

import jax
import jax.numpy as jnp
from jax import lax
from jax.experimental import pallas as pl
from jax.experimental.pallas import tpu as pltpu
from jax.experimental.pallas import tpu_sc as plsc

_VOCAB = 1000000
_EMBED = 32
_FIELDS = 26
_FPAD = 28
_BATCH = 16384
_HIDDEN = 128
_K = _FPAD * _EMBED
_TOT = _BATCH * _FPAD
_NW = 32
_PER_W = _TOT // _NW
_CHUNK = 1792
_NCH = _PER_W // _CHUNK

_mesh = plsc.VectorSubcoreMesh(core_axis_name="c", subcore_axis_name="s")


@functools.partial(
    pl.kernel,
    mesh=_mesh,
    out_type=jax.ShapeDtypeStruct((_TOT, _EMBED), jnp.float32),
    scratch_types=[
        pltpu.VMEM((2, _CHUNK), jnp.int32),
        pltpu.VMEM((2, _CHUNK, _EMBED), jnp.float32),
        pltpu.SemaphoreType.DMA,
        pltpu.SemaphoreType.DMA,
        pltpu.SemaphoreType.DMA,
        pltpu.SemaphoreType.DMA,
    ],
    compiler_params=pltpu.CompilerParams(use_tc_tiling_on_sc=False),
)
def _sc_gather(idx_hbm, table_hbm, out_hbm, idx_v, rows_v, sg0, sg1, sw0, sw1):
    wid = lax.axis_index("s") * 2 + lax.axis_index("c")
    base = wid * _PER_W
    semg = (sg0, sg1)
    semw = (sw0, sw1)
    gather = [None, None]
    write = [None, None]

    pltpu.sync_copy(idx_hbm.at[pl.ds(base, _CHUNK)], idx_v.at[0])
    gather[0] = pltpu.async_copy(table_hbm.at[idx_v.at[0]], rows_v.at[0], semg[0])
    for i in range(_NCH):
        s = i % 2
        o = (i + 1) % 2
        if i + 1 < _NCH:
            off = base + (i + 1) * _CHUNK
            pltpu.sync_copy(idx_hbm.at[pl.ds(off, _CHUNK)], idx_v.at[o])
            if write[o] is not None:
                write[o].wait()
            gather[o] = pltpu.async_copy(
                table_hbm.at[idx_v.at[o]], rows_v.at[o], semg[o]
            )
        gather[s].wait()
        write[s] = pltpu.async_copy(
            rows_v.at[s], out_hbm.at[pl.ds(base + i * _CHUNK, _CHUNK)], semw[s]
        )
    write[0].wait()
    write[1].wait()


def _mm_body(flat_ref, w_ref, b_ref, o_ref):
    o_ref[...] = (
        jnp.dot(flat_ref[...], w_ref[...], preferred_element_type=jnp.float32)
        + b_ref[...]
    )


_BM = 1024


def _tc_matmul(flat, Wp, b2):
    return pl.pallas_call(
        _mm_body,
        grid=(_BATCH // _BM,),
        in_specs=[
            pl.BlockSpec((_BM, _K), lambda i: (i, 0)),
            pl.BlockSpec((_K, _HIDDEN), lambda i: (0, 0)),
            pl.BlockSpec((1, _HIDDEN), lambda i: (0, 0)),
        ],
        out_specs=pl.BlockSpec((_BM, _HIDDEN), lambda i: (i, 0)),
        out_shape=jax.ShapeDtypeStruct((_BATCH, _HIDDEN), jnp.float32),
    )(flat, Wp, b2)


def kernel(X, table, W, b):
    idx = jnp.pad(X, ((0, 0), (0, _FPAD - _FIELDS))).reshape(-1)
    rows = _sc_gather(idx, table)
    flat = rows.reshape(_BATCH, _K)
    Wp = jnp.concatenate(
        [W, jnp.zeros((_K - _FIELDS * _EMBED, _HIDDEN), jnp.float32)], axis=0
    )
    return _tc_matmul(flat, Wp, b.reshape(1, _HIDDEN))
